# baseline (speedup 1.0000x reference)
import functools

import jax
import jax.numpy as jnp
from jax import lax
from jax.experimental import pallas as pl
from jax.experimental.pallas import tpu as pltpu
from jax.experimental.pallas import tpu_sc as plsc

B, C = 4096, 1000
NC, NS, L = 2, 16, 16
NW = NC * NS
SC_ROWS = 1024
ROWS_PER_W = SC_ROWS // NW
CH = 16
NCHUNK = ROWS_PER_W // CH
MARGIN = 1.0
NEG_INF = float("-inf")

TC_BLK = 256
TC_BLK0 = SC_ROWS // TC_BLK
TC_NBLK = (B - SC_ROWS) // TC_BLK

_mesh = plsc.VectorSubcoreMesh(core_axis_name="c", subcore_axis_name="s")


@functools.partial(
    pl.kernel,
    out_type=jax.ShapeDtypeStruct((NW, L), jnp.float32),
    mesh=_mesh,
    compiler_params=pltpu.CompilerParams(needs_layout_passes=False),
    scratch_types=(
        [pltpu.VMEM((CH, C), jnp.float32)] * NCHUNK
        + [
            pltpu.VMEM((ROWS_PER_W,), jnp.int32),
            pltpu.VMEM((L,), jnp.float32),
        ]
        + [pltpu.SemaphoreType.DMA] * NCHUNK
    ),
)
def _hinge_sc(x_hbm, tgt_hbm, out_hbm, *refs):
    NBUF = NCHUNK
    bufs = refs[:NBUF]
    tgtv = refs[NBUF]
    stage = refs[NBUF + 1]
    sems = refs[NBUF + 2:]
    cid = lax.axis_index("c")
    sid = lax.axis_index("s")
    wid = sid * NC + cid
    base_row = wid * ROWS_PER_W

    pltpu.sync_copy(tgt_hbm.at[pl.ds(wid * ROWS_PER_W, ROWS_PER_W)], tgtv)

    lane = lax.iota(jnp.int32, L)
    neg_inf_v = lax.broadcast(jnp.float32(NEG_INF), (L,))

    copies = [
        pltpu.async_copy(
            x_hbm.at[pl.ds(base_row + p * CH, CH), :], bufs[p], sems[p])
        for p in range(NCHUNK)
    ]

    acc = lax.broadcast(jnp.float32(0.0), (L,))
    for ch in range(NCHUNK):
        copies[ch].wait()
        buf = bufs[ch]
        tcol = tgtv[pl.ds(ch * CH, L)]
        pos = plsc.load_gather(buf, [lane, tcol])
        plsc.store_scatter(buf, [lane, tcol], neg_inf_v)

        def row_body(r, rmax, buf=buf):
            m = buf[r, pl.ds(0, L)]
            for cc in range(1, C // L):
                m = jnp.maximum(m, buf[r, pl.ds(cc * L, L)])
            m = jnp.maximum(m, buf[r, pl.ds(C - L, L)])
            s = jnp.max(m)
            return jnp.where(lane == r, s, rmax)

        rmax = lax.fori_loop(0, CH, row_body, neg_inf_v)
        acc = acc + jnp.maximum(jnp.float32(0.0),
                                jnp.float32(MARGIN) - pos + rmax)

    stage[...] = acc
    pltpu.sync_copy(stage, out_hbm.at[wid])


def _hinge_tc_body(x_ref, t_ref, o_ref):
    i = pl.program_id(0)
    x = x_ref[...]
    t = t_ref[...][:, None]
    colid = lax.broadcasted_iota(jnp.int32, (TC_BLK, C), 1)
    is_t = colid == t
    pos = jnp.sum(jnp.where(is_t, x, jnp.float32(0.0)), axis=1)
    mneg = jnp.max(jnp.where(is_t, jnp.float32(NEG_INF), x), axis=1)
    losses = jnp.maximum(jnp.float32(0.0),
                         jnp.float32(MARGIN) - pos + mneg)
    part = jnp.sum(losses).reshape(1, 1)

    @pl.when(i == 0)
    def _():
        o_ref[...] = part

    @pl.when(i > 0)
    def _():
        o_ref[...] += part


_hinge_tc = pl.pallas_call(
    _hinge_tc_body,
    grid=(TC_NBLK,),
    in_specs=[pl.BlockSpec((TC_BLK, C), lambda i: (i + TC_BLK0, 0)),
              pl.BlockSpec((TC_BLK,), lambda i: (i + TC_BLK0,))],
    out_specs=pl.BlockSpec((1, 1), lambda i: (0, 0)),
    out_shape=jax.ShapeDtypeStruct((1, 1), jnp.float32),
)


def _reduce_tc_body(p_ref, t_ref, o_ref):
    o_ref[...] = ((jnp.sum(p_ref[...]) + t_ref[0, 0])
                  * jnp.float32(1.0 / B)).reshape(1, 1)


_reduce_tc = pl.pallas_call(
    _reduce_tc_body,
    out_shape=jax.ShapeDtypeStruct((1, 1), jnp.float32),
    in_specs=[pl.BlockSpec(memory_space=pltpu.VMEM),
              pl.BlockSpec(memory_space=pltpu.VMEM)],
    out_specs=pl.BlockSpec(memory_space=pltpu.VMEM),
)


def kernel(input, target):
    partials = _hinge_sc(input, target)
    tc_part = _hinge_tc(input, target)
    return _reduce_tc(partials, tc_part)[0, 0]


# speedup vs baseline: 1.0274x; 1.0274x over previous
import functools

import jax
import jax.numpy as jnp
from jax import lax
from jax.experimental import pallas as pl
from jax.experimental.pallas import tpu as pltpu
from jax.experimental.pallas import tpu_sc as plsc

B, C = 4096, 1000
NC, NS, L = 2, 16, 16
NW = NC * NS
ROWS_PER_W = B // NW
CH = 16
NCHUNK = ROWS_PER_W // CH
MARGIN = 1.0
NEG_INF = float("-inf")

_mesh = plsc.VectorSubcoreMesh(core_axis_name="c", subcore_axis_name="s")


@functools.partial(
    pl.kernel,
    out_type=jax.ShapeDtypeStruct((NW, L), jnp.float32),
    mesh=_mesh,
    compiler_params=pltpu.CompilerParams(needs_layout_passes=False),
    scratch_types=(
        [pltpu.VMEM((CH, C), jnp.float32)] * 4
        + [
            pltpu.VMEM((ROWS_PER_W,), jnp.int32),
            pltpu.VMEM((L,), jnp.float32),
        ]
        + [pltpu.SemaphoreType.DMA] * 4
    ),
)
def _hinge_sc(x_hbm, tgt_hbm, out_hbm, *refs):
    NBUF = 4
    bufs = refs[:NBUF]
    tgtv = refs[NBUF]
    stage = refs[NBUF + 1]
    sems = refs[NBUF + 2:]
    cid = lax.axis_index("c")
    sid = lax.axis_index("s")
    wid = sid * NC + cid
    base_row = wid * ROWS_PER_W

    pltpu.sync_copy(tgt_hbm.at[pl.ds(wid * ROWS_PER_W, ROWS_PER_W)], tgtv)

    lane = lax.iota(jnp.int32, L)
    neg_inf_v = lax.broadcast(jnp.float32(NEG_INF), (L,))

    copies = [None] * NBUF
    for p in range(NBUF - 1):
        copies[p] = pltpu.async_copy(
            x_hbm.at[pl.ds(base_row + p * CH, CH), :], bufs[p], sems[p])

    acc = lax.broadcast(jnp.float32(0.0), (L,))
    for ch in range(NCHUNK):
        par = ch % NBUF
        copies[par].wait()
        if ch + NBUF - 1 < NCHUNK:
            npar = (ch + NBUF - 1) % NBUF
            copies[npar] = pltpu.async_copy(
                x_hbm.at[pl.ds(base_row + (ch + NBUF - 1) * CH, CH), :],
                bufs[npar], sems[npar])
        buf = bufs[par]
        tcol = tgtv[pl.ds(ch * CH, L)]
        pos = plsc.load_gather(buf, [lane, tcol])
        plsc.store_scatter(buf, [lane, tcol], neg_inf_v)

        def row_body(r, rmax, buf=buf):
            m = buf[r, pl.ds(0, L)]
            for cc in range(1, C // L):
                m = jnp.maximum(m, buf[r, pl.ds(cc * L, L)])
            m = jnp.maximum(m, buf[r, pl.ds(C - L, L)])
            s = jnp.max(m)
            return jnp.where(lane == r, s, rmax)

        rmax = lax.fori_loop(0, CH, row_body, neg_inf_v)
        acc = acc + jnp.maximum(jnp.float32(0.0),
                                jnp.float32(MARGIN) - pos + rmax)

    stage[...] = acc
    pltpu.sync_copy(stage, out_hbm.at[wid])


def _reduce_tc_body(p_ref, o_ref):
    o_ref[...] = (jnp.sum(p_ref[...]) * jnp.float32(1.0 / B)).reshape(1, 1)


_reduce_tc = pl.pallas_call(
    _reduce_tc_body,
    out_shape=jax.ShapeDtypeStruct((1, 1), jnp.float32),
    in_specs=[pl.BlockSpec(memory_space=pltpu.VMEM)],
    out_specs=pl.BlockSpec(memory_space=pltpu.VMEM),
)


def kernel(input, target):
    partials = _hinge_sc(input, target)
    return _reduce_tc(partials)[0, 0]
